# baseline (speedup 1.0000x reference)
import functools

import jax
import jax.numpy as jnp
from jax.experimental import pallas as pl
from jax.experimental.pallas import tpu as pltpu

_BM = 256


def _hollow_matmul_kernel(w_ref, x_ref, o_ref, *, bm):
    i = pl.program_id(0)
    w = w_ref[...]
    r = jax.lax.broadcasted_iota(jnp.int32, w.shape, 0)
    c = jax.lax.broadcasted_iota(jnp.int32, w.shape, 1)
    w = jnp.where(c == r + i * bm, jnp.zeros_like(w), w)
    o_ref[...] = jnp.dot(w, x_ref[...], preferred_element_type=jnp.float32)


def kernel(weight, x):
    n, n2 = weight.shape
    assert n == n2
    d = x.shape[1]
    bm = _BM
    assert n % bm == 0

    out = pl.pallas_call(
        functools.partial(_hollow_matmul_kernel, bm=bm),
        grid=(n // bm,),
        in_specs=[
            pl.BlockSpec((bm, n), lambda i: (i, 0)),
            pl.BlockSpec((n, d), lambda i: (0, 0)),
        ],
        out_specs=pl.BlockSpec((bm, d), lambda i: (i, 0)),
        out_shape=jax.ShapeDtypeStruct((n, d), jnp.float32),
        compiler_params=pltpu.CompilerParams(
            dimension_semantics=("parallel",),
            vmem_limit_bytes=64 * 1024 * 1024,
        ),
    )(weight, x)
    return weight, out


# speedup vs baseline: 1.0597x; 1.0597x over previous
import functools

import jax
import jax.numpy as jnp
from jax.experimental import pallas as pl
from jax.experimental.pallas import tpu as pltpu

_BM = 1024


def _hollow_matmul_kernel(w_ref, x_ref, o_ref, *, bm):
    i = pl.program_id(0)
    w = w_ref[...]
    r = jax.lax.broadcasted_iota(jnp.int32, w.shape, 0)
    c = jax.lax.broadcasted_iota(jnp.int32, w.shape, 1)
    w = jnp.where(c == r + i * bm, jnp.zeros_like(w), w)
    o_ref[...] = jnp.dot(w, x_ref[...], preferred_element_type=jnp.float32)


def kernel(weight, x):
    n, n2 = weight.shape
    assert n == n2
    d = x.shape[1]
    bm = _BM
    assert n % bm == 0

    out = pl.pallas_call(
        functools.partial(_hollow_matmul_kernel, bm=bm),
        grid=(n // bm,),
        in_specs=[
            pl.BlockSpec((bm, n), lambda i: (i, 0)),
            pl.BlockSpec((n, d), lambda i: (0, 0)),
        ],
        out_specs=pl.BlockSpec((bm, d), lambda i: (i, 0)),
        out_shape=jax.ShapeDtypeStruct((n, d), jnp.float32),
        compiler_params=pltpu.CompilerParams(
            dimension_semantics=("parallel",),
            vmem_limit_bytes=64 * 1024 * 1024,
        ),
    )(weight, x)
    return weight, out
